# baseline (speedup 1.0000x reference)
import functools

import jax
import jax.numpy as jnp
from jax import lax
from jax.experimental import pallas as pl
from jax.experimental.pallas import tpu as pltpu
from jax.experimental.pallas import tpu_sc as plsc

_LANES = 16


@functools.lru_cache(maxsize=None)
def _make_sc_gather(B, n, d, CH, NBUF):
    info = plsc.get_sparse_core_info()
    NC, NS = info.num_cores, info.num_subcores
    NW = NC * NS
    assert B % NW == 0 and n % CH == 0 and n % _LANES == 0
    n_b = B // NW
    NCH = n // CH
    assert NCH % NBUF == 0

    mesh = plsc.VectorSubcoreMesh(core_axis_name="c", subcore_axis_name="s")
    out_type = jax.ShapeDtypeStruct((B * n, d), jnp.float32)
    scratch = [pltpu.VMEM((n,), jnp.int32)]
    scratch += [pltpu.VMEM((CH, d), jnp.float32) for _ in range(NBUF)]
    scratch += [pltpu.SemaphoreType.DMA for _ in range(2 * NBUF)]

    @functools.partial(pl.kernel, mesh=mesh, out_type=out_type,
                       scratch_types=scratch)
    def sc_kernel(emb, order, out, idx, *rest):
        bufs = rest[:NBUF]
        gsem = rest[NBUF:2 * NBUF]
        ssem = rest[2 * NBUF:]
        wid = lax.axis_index("s") * NC + lax.axis_index("c")

        for kk in range(n_b):
            b = wid * n_b + kk
            base = b * n

            pltpu.sync_copy(order, idx)
            for j in range(n // _LANES):
                sl = pl.ds(j * _LANES, _LANES)
                idx[sl] = idx[sl] + base

            def gather_start(c, s):
                pltpu.async_copy(emb.at[idx.at[pl.ds(c * CH, CH)]],
                                 bufs[s], gsem[s])

            def gather_wait(c, s):
                pltpu.make_async_copy(emb.at[idx.at[pl.ds(c * CH, CH)]],
                                      bufs[s], gsem[s]).wait()

            def store_start(c, s):
                pltpu.async_copy(bufs[s], out.at[pl.ds(base + c * CH, CH)],
                                 ssem[s])

            def store_wait(c, s):
                pltpu.make_async_copy(bufs[s],
                                      out.at[pl.ds(base + c * CH, CH)],
                                      ssem[s]).wait()

            for s in range(NBUF):
                gather_start(s, s)

            def body(i, _):
                for s in range(NBUF):
                    c = i * NBUF + s
                    gather_wait(c, s)
                    store_start(c, s)
                for s in range(NBUF):
                    c2 = (i + 1) * NBUF + s

                    def refill(s=s, c2=c2):
                        store_wait(c2 - NBUF, s)
                        gather_start(c2, s)

                    pl.when(c2 < NCH)(refill)
                return 0

            lax.fori_loop(0, NCH // NBUF, body, 0)

            for s in range(NBUF):
                store_wait(NCH - NBUF + s, s)

    return sc_kernel


def kernel(embeddings, order):
    B, n, d = embeddings.shape
    order_i = order.astype(jnp.int32)
    f = _make_sc_gather(B, n, d, 64, 2)
    out = f(embeddings.reshape(B * n, d), order_i)
    return (out.reshape(B, n, d), None)


# speedup vs baseline: 1.0315x; 1.0315x over previous
import functools

import jax
import jax.numpy as jnp
from jax import lax
from jax.experimental import pallas as pl
from jax.experimental.pallas import tpu as pltpu
from jax.experimental.pallas import tpu_sc as plsc

_LANES = 16


@functools.lru_cache(maxsize=None)
def _make_sc_gather(B, n, d, CH, NBUF):
    info = plsc.get_sparse_core_info()
    NC, NS = info.num_cores, info.num_subcores
    NW = NC * NS
    assert B % NW == 0 and n % CH == 0 and n % _LANES == 0
    n_b = B // NW
    NCH = n // CH
    assert NCH % NBUF == 0

    mesh = plsc.VectorSubcoreMesh(core_axis_name="c", subcore_axis_name="s")
    out_type = jax.ShapeDtypeStruct((B * n, d), jnp.float32)
    scratch = [pltpu.VMEM((n,), jnp.int32)]
    scratch += [pltpu.VMEM((CH, d), jnp.float32) for _ in range(NBUF)]
    scratch += [pltpu.SemaphoreType.DMA for _ in range(2 * NBUF)]

    @functools.partial(pl.kernel, mesh=mesh, out_type=out_type,
                       scratch_types=scratch)
    def sc_kernel(emb, order, out, idx, *rest):
        bufs = rest[:NBUF]
        gsem = rest[NBUF:2 * NBUF]
        ssem = rest[2 * NBUF:]
        wid = lax.axis_index("s") * NC + lax.axis_index("c")

        for kk in range(n_b):
            b = wid * n_b + kk
            base = b * n

            pltpu.sync_copy(order, idx)
            for j in range(n // _LANES):
                sl = pl.ds(j * _LANES, _LANES)
                idx[sl] = idx[sl] + base

            def gather_start(c, s):
                pltpu.async_copy(emb.at[idx.at[pl.ds(c * CH, CH)]],
                                 bufs[s], gsem[s])

            def gather_wait(c, s):
                pltpu.make_async_copy(emb.at[idx.at[pl.ds(c * CH, CH)]],
                                      bufs[s], gsem[s]).wait()

            def store_start(c, s):
                pltpu.async_copy(bufs[s], out.at[pl.ds(base + c * CH, CH)],
                                 ssem[s])

            def store_wait(c, s):
                pltpu.make_async_copy(bufs[s],
                                      out.at[pl.ds(base + c * CH, CH)],
                                      ssem[s]).wait()

            for s in range(NBUF):
                gather_start(s, s)

            def body(i, _):
                for s in range(NBUF):
                    c = i * NBUF + s
                    gather_wait(c, s)
                    store_start(c, s)
                for s in range(NBUF):
                    c2 = (i + 1) * NBUF + s

                    def refill(s=s, c2=c2):
                        store_wait(c2 - NBUF, s)
                        gather_start(c2, s)

                    pl.when(c2 < NCH)(refill)
                return 0

            lax.fori_loop(0, NCH // NBUF, body, 0)

            for s in range(NBUF):
                store_wait(NCH - NBUF + s, s)

    return sc_kernel


def kernel(embeddings, order):
    B, n, d = embeddings.shape
    order_i = order.astype(jnp.int32)
    f = _make_sc_gather(B, n, d, 16, 8)
    out = f(embeddings.reshape(B * n, d), order_i)
    return (out.reshape(B, n, d), None)
